# baseline (speedup 1.0000x reference)
import numpy as np
import jax
import jax.numpy as jnp
from jax import lax
from jax.experimental import pallas as pl
from jax.experimental.pallas import tpu as pltpu
from jax.experimental.pallas import tpu_sc as plsc

VOCAB = 100000
D = 1024
HD = D // 2
B = 4
L = 2048

NC = 2
NS = 16
NW = NC * NS
P_PER_W = L // NW
CHUNK = 8
NCHB = P_PER_W // CHUNK
NT = B * NCHB
LANES = 16
VECS = D // LANES
KG = HD // LANES
NG = 4
NWB = 2

SCALE = float(np.sqrt(D))


STRIDE = 8
NCHAIN = P_PER_W // STRIDE
NBASE = L // STRIDE


def _pos_constants():
    k = np.arange(HD, dtype=np.float64)
    rates = 1.0 / (10000.0 ** (k / HD))
    base_p = (np.arange(NBASE, dtype=np.float64) * STRIDE)[:, None]
    ang = base_p * rates[None, :]
    base_rows = np.concatenate([np.sin(ang), np.cos(ang)], axis=-1)
    coef_row = np.concatenate([np.cos(rates), np.sin(rates)])[None, :]
    return np.concatenate([base_rows, coef_row], axis=0).astype(np.float32)


_POS_CST = _pos_constants()


def _sc_body(x_hbm, table_hbm, cst_hbm, out_hbm,
             idx_all, pos_v, coef_v, stage_v, g0, g1, g2, g3, w0, w1,
             csem, ssem, xsem, gs0, gs1, gs2, gs3, ws0, ws1):
    c = lax.axis_index("c")
    s = lax.axis_index("s")
    wid = s * NC + c
    base_p = wid * P_PER_W

    xds = [pltpu.async_copy(x_hbm.at[b, pl.ds(base_p, P_PER_W)],
                            idx_all.at[pl.ds(b * P_PER_W, P_PER_W)], xsem)
           for b in range(B)]
    for d in xds:
        d.wait()

    gbufs = [g0, g1, g2, g3]
    wbufs = [w0, w1]
    gsem = [gs0, gs1, gs2, gs3]
    wsem = [ws0, ws1]
    gd = [None] * NG
    wd = [None] * NWB

    def start_gather(t):
        slot = t % NG
        gd[slot] = pltpu.async_copy(
            table_hbm.at[idx_all.at[pl.ds(t * CHUNK, CHUNK)]],
            gbufs[slot], gsem[slot])

    for t in range(NG):
        start_gather(t)

    st_d = pltpu.async_copy(
        cst_hbm.at[pl.ds(wid * NCHAIN, NCHAIN)], stage_v, ssem)
    cf_d = pltpu.async_copy(cst_hbm.at[pl.ds(NBASE, 1)], coef_v, csem)
    st_d.wait()
    cf_d.wait()

    def gen_chain(ch):
        @plsc.parallel_loop(0, KG, unroll=4)
        def _(i):
            col = i * LANES
            cc = coef_v[0, pl.ds(col, LANES)]
            cs = coef_v[0, pl.ds(HD + col, LANES)]
            ps = stage_v[ch, pl.ds(col, LANES)]
            pc = stage_v[ch, pl.ds(HD + col, LANES)]
            pos_v[ch * STRIDE, pl.ds(col, LANES)] = ps
            pos_v[ch * STRIDE, pl.ds(HD + col, LANES)] = pc
            for st in range(1, STRIDE):
                ps, pc = ps * cc + pc * cs, pc * cc - ps * cs
                pos_v[ch * STRIDE + st, pl.ds(col, LANES)] = ps
                pos_v[ch * STRIDE + st, pl.ds(HD + col, LANES)] = pc

    for t in range(NT):
        if t < NCHB:
            gen_chain(t)
        gslot = t % NG
        wslot = t % NWB
        gd[gslot].wait()
        if wd[wslot] is not None:
            wd[wslot].wait()
        b, h = divmod(t, NCHB)
        rg = gbufs[gslot]
        rw = wbufs[wslot]
        prow = h * CHUNK

        @plsc.parallel_loop(0, CHUNK * VECS, unroll=8)
        def _(i):
            r = i // VECS
            col = (i % VECS) * LANES
            e = rg[r, pl.ds(col, LANES)]
            p = pos_v[prow + r, pl.ds(col, LANES)]
            rw[r, pl.ds(col, LANES)] = e * SCALE + p

        row0 = b * L + base_p + h * CHUNK
        wd[wslot] = pltpu.async_copy(rw, out_hbm.at[pl.ds(row0, CHUNK)],
                                     wsem[wslot])
        nt = t + NG
        if nt < NT:
            start_gather(nt)

    wd[0].wait()
    wd[1].wait()


def kernel(x, table):
    cst = jnp.asarray(_POS_CST)

    mesh = plsc.VectorSubcoreMesh(
        core_axis_name="c", subcore_axis_name="s", num_cores=NC, num_subcores=NS
    )
    k = pl.kernel(
        _sc_body,
        out_type=jax.ShapeDtypeStruct((B * L, D), jnp.float32),
        mesh=mesh,
        scratch_types=[
            pltpu.VMEM((B * P_PER_W,), jnp.int32),
            pltpu.VMEM((P_PER_W, D), jnp.float32),
            pltpu.VMEM((1, D), jnp.float32),
            pltpu.VMEM((NCHAIN, D), jnp.float32),
            pltpu.VMEM((CHUNK, D), jnp.float32),
            pltpu.VMEM((CHUNK, D), jnp.float32),
            pltpu.VMEM((CHUNK, D), jnp.float32),
            pltpu.VMEM((CHUNK, D), jnp.float32),
            pltpu.VMEM((CHUNK, D), jnp.float32),
            pltpu.VMEM((CHUNK, D), jnp.float32),
            pltpu.SemaphoreType.DMA,
            pltpu.SemaphoreType.DMA,
            pltpu.SemaphoreType.DMA,
            pltpu.SemaphoreType.DMA,
            pltpu.SemaphoreType.DMA,
            pltpu.SemaphoreType.DMA,
            pltpu.SemaphoreType.DMA,
            pltpu.SemaphoreType.DMA,
            pltpu.SemaphoreType.DMA,
        ],
    )
    out = k(x, table, cst)
    return out.reshape(B, L, D)


# speedup vs baseline: 1.0203x; 1.0203x over previous
import numpy as np
import jax
import jax.numpy as jnp
from jax import lax
from jax.experimental import pallas as pl
from jax.experimental.pallas import tpu as pltpu
from jax.experimental.pallas import tpu_sc as plsc

VOCAB = 100000
D = 1024
HD = D // 2
B = 4
L = 2048

NC = 2
NS = 16
NW = NC * NS
P_PER_W = L // NW
CHUNK = 8
NCHB = P_PER_W // CHUNK
NT = B * NCHB
LANES = 16
VECS = D // LANES
KG = HD // LANES
NG = 4
NWB = 3

SCALE = float(np.sqrt(D))


STRIDE = 8
NCHAIN = P_PER_W // STRIDE
NBASE = L // STRIDE


def _pos_constants():
    k = np.arange(HD, dtype=np.float64)
    rates = 1.0 / (10000.0 ** (k / HD))
    base_p = (np.arange(NBASE, dtype=np.float64) * STRIDE)[:, None]
    ang = base_p * rates[None, :]
    base_rows = np.concatenate([np.sin(ang), np.cos(ang)], axis=-1)
    coef_row = np.concatenate([np.cos(rates), np.sin(rates)])[None, :]
    return np.concatenate([base_rows, coef_row], axis=0).astype(np.float32)


_POS_CST = _pos_constants()


def _sc_body(x_hbm, table_hbm, cst_hbm, out_hbm,
             idx_all, pos_v, coef_v, stage_v, g0, g1, g2, g3, w0, w1,
             csem, ssem, xsem, gs0, gs1, gs2, gs3, ws0, ws1, ws2):
    c = lax.axis_index("c")
    s = lax.axis_index("s")
    wid = s * NC + c
    base_p = wid * P_PER_W

    xds = [pltpu.async_copy(x_hbm.at[b, pl.ds(base_p, P_PER_W)],
                            idx_all.at[pl.ds(b * P_PER_W, P_PER_W)], xsem)
           for b in range(B)]
    for d in xds:
        d.wait()

    gbufs = [g0, g1, g2, g3]
    wbufs = [w0, w1, stage_v]
    gsem = [gs0, gs1, gs2, gs3]
    wsem = [ws0, ws1, ws2]
    gd = [None] * NG
    wd = [None] * NWB

    def start_gather(t):
        slot = t % NG
        gd[slot] = pltpu.async_copy(
            table_hbm.at[idx_all.at[pl.ds(t * CHUNK, CHUNK)]],
            gbufs[slot], gsem[slot])

    for t in range(NG):
        start_gather(t)

    st_d = pltpu.async_copy(
        cst_hbm.at[pl.ds(wid * NCHAIN, NCHAIN)], stage_v, ssem)
    cf_d = pltpu.async_copy(cst_hbm.at[pl.ds(NBASE, 1)], coef_v, csem)
    st_d.wait()
    cf_d.wait()

    @plsc.parallel_loop(0, NCHAIN * KG, unroll=4)
    def _(i):
        ch = i // KG
        col = (i % KG) * LANES
        cc = coef_v[0, pl.ds(col, LANES)]
        cs = coef_v[0, pl.ds(HD + col, LANES)]
        ps = stage_v[ch, pl.ds(col, LANES)]
        pc = stage_v[ch, pl.ds(HD + col, LANES)]
        pos_v[ch * STRIDE, pl.ds(col, LANES)] = ps
        pos_v[ch * STRIDE, pl.ds(HD + col, LANES)] = pc
        for st in range(1, STRIDE):
            ps, pc = ps * cc + pc * cs, pc * cc - ps * cs
            pos_v[ch * STRIDE + st, pl.ds(col, LANES)] = ps
            pos_v[ch * STRIDE + st, pl.ds(HD + col, LANES)] = pc

    for t in range(NT):
        gslot = t % NG
        wslot = t % NWB
        gd[gslot].wait()
        if wd[wslot] is not None:
            wd[wslot].wait()
        b, h = divmod(t, NCHB)
        rg = gbufs[gslot]
        rw = wbufs[wslot]
        prow = h * CHUNK

        @plsc.parallel_loop(0, CHUNK * VECS, unroll=8)
        def _(i):
            r = i // VECS
            col = (i % VECS) * LANES
            e = rg[r, pl.ds(col, LANES)]
            p = pos_v[prow + r, pl.ds(col, LANES)]
            rw[r, pl.ds(col, LANES)] = e * SCALE + p

        row0 = b * L + base_p + h * CHUNK
        wd[wslot] = pltpu.async_copy(rw, out_hbm.at[pl.ds(row0, CHUNK)],
                                     wsem[wslot])
        nt = t + NG
        if nt < NT:
            start_gather(nt)

    for d in wd:
        if d is not None:
            d.wait()


def kernel(x, table):
    cst = jnp.asarray(_POS_CST)

    mesh = plsc.VectorSubcoreMesh(
        core_axis_name="c", subcore_axis_name="s", num_cores=NC, num_subcores=NS
    )
    k = pl.kernel(
        _sc_body,
        out_type=jax.ShapeDtypeStruct((B * L, D), jnp.float32),
        mesh=mesh,
        scratch_types=[
            pltpu.VMEM((B * P_PER_W,), jnp.int32),
            pltpu.VMEM((P_PER_W, D), jnp.float32),
            pltpu.VMEM((1, D), jnp.float32),
            pltpu.VMEM((NCHAIN, D), jnp.float32),
            pltpu.VMEM((CHUNK, D), jnp.float32),
            pltpu.VMEM((CHUNK, D), jnp.float32),
            pltpu.VMEM((CHUNK, D), jnp.float32),
            pltpu.VMEM((CHUNK, D), jnp.float32),
            pltpu.VMEM((CHUNK, D), jnp.float32),
            pltpu.VMEM((CHUNK, D), jnp.float32),
            pltpu.SemaphoreType.DMA,
            pltpu.SemaphoreType.DMA,
            pltpu.SemaphoreType.DMA,
            pltpu.SemaphoreType.DMA,
            pltpu.SemaphoreType.DMA,
            pltpu.SemaphoreType.DMA,
            pltpu.SemaphoreType.DMA,
            pltpu.SemaphoreType.DMA,
            pltpu.SemaphoreType.DMA,
            pltpu.SemaphoreType.DMA,
        ],
    )
    out = k(x, table, cst)
    return out.reshape(B, L, D)


# speedup vs baseline: 1.0337x; 1.0132x over previous
import numpy as np
import jax
import jax.numpy as jnp
from jax import lax
from jax.experimental import pallas as pl
from jax.experimental.pallas import tpu as pltpu
from jax.experimental.pallas import tpu_sc as plsc

VOCAB = 100000
D = 1024
HD = D // 2
B = 4
L = 2048

NC = 2
NS = 16
NW = NC * NS
P_PER_W = L // NW
CHUNK = 8
NCHB = P_PER_W // CHUNK
NT = B * NCHB
LANES = 16
VECS = D // LANES
KG = HD // LANES
NG = 5
NWB = 2

SCALE = float(np.sqrt(D))


STRIDE = 8
NCHAIN = P_PER_W // STRIDE
NBASE = L // STRIDE


def _pos_constants():
    k = np.arange(HD, dtype=np.float64)
    rates = 1.0 / (10000.0 ** (k / HD))
    base_p = (np.arange(NBASE, dtype=np.float64) * STRIDE)[:, None]
    ang = base_p * rates[None, :]
    base_rows = np.concatenate([np.sin(ang), np.cos(ang)], axis=-1)
    coef_row = np.concatenate([np.cos(rates), np.sin(rates)])[None, :]
    return np.concatenate([base_rows, coef_row], axis=0).astype(np.float32)


_POS_CST = _pos_constants()


def _sc_body(x_hbm, table_hbm, cst_hbm, out_hbm,
             idx_all, pos_v, coef_v, stage_v, g0, g1, g2, g3, w0, w1,
             csem, ssem, xsem, gs0, gs1, gs2, gs3, gs4, ws0, ws1):
    c = lax.axis_index("c")
    s = lax.axis_index("s")
    wid = s * NC + c
    base_p = wid * P_PER_W

    xds = [pltpu.async_copy(x_hbm.at[b, pl.ds(base_p, P_PER_W)],
                            idx_all.at[pl.ds(b * P_PER_W, P_PER_W)], xsem)
           for b in range(B)]
    for d in xds:
        d.wait()

    gbufs = [g0, g1, g2, g3, stage_v]
    wbufs = [w0, w1]
    gsem = [gs0, gs1, gs2, gs3, gs4]
    wsem = [ws0, ws1]
    gd = [None] * NG
    wd = [None] * NWB

    def start_gather(t):
        slot = t % NG
        gd[slot] = pltpu.async_copy(
            table_hbm.at[idx_all.at[pl.ds(t * CHUNK, CHUNK)]],
            gbufs[slot], gsem[slot])

    for t in range(NG - 1):
        start_gather(t)

    st_d = pltpu.async_copy(
        cst_hbm.at[pl.ds(wid * NCHAIN, NCHAIN)], stage_v, ssem)
    cf_d = pltpu.async_copy(cst_hbm.at[pl.ds(NBASE, 1)], coef_v, csem)
    st_d.wait()
    cf_d.wait()

    @plsc.parallel_loop(0, NCHAIN * KG, unroll=4)
    def _(i):
        ch = i // KG
        col = (i % KG) * LANES
        cc = coef_v[0, pl.ds(col, LANES)]
        cs = coef_v[0, pl.ds(HD + col, LANES)]
        ps = stage_v[ch, pl.ds(col, LANES)]
        pc = stage_v[ch, pl.ds(HD + col, LANES)]
        pos_v[ch * STRIDE, pl.ds(col, LANES)] = ps
        pos_v[ch * STRIDE, pl.ds(HD + col, LANES)] = pc
        for st in range(1, STRIDE):
            ps, pc = ps * cc + pc * cs, pc * cc - ps * cs
            pos_v[ch * STRIDE + st, pl.ds(col, LANES)] = ps
            pos_v[ch * STRIDE + st, pl.ds(HD + col, LANES)] = pc

    start_gather(NG - 1)

    for t in range(NT):
        gslot = t % NG
        wslot = t % NWB
        gd[gslot].wait()
        if wd[wslot] is not None:
            wd[wslot].wait()
        b, h = divmod(t, NCHB)
        rg = gbufs[gslot]
        rw = wbufs[wslot]
        prow = h * CHUNK

        @plsc.parallel_loop(0, CHUNK * VECS, unroll=8)
        def _(i):
            r = i // VECS
            col = (i % VECS) * LANES
            e = rg[r, pl.ds(col, LANES)]
            p = pos_v[prow + r, pl.ds(col, LANES)]
            rw[r, pl.ds(col, LANES)] = e * SCALE + p

        row0 = b * L + base_p + h * CHUNK
        wd[wslot] = pltpu.async_copy(rw, out_hbm.at[pl.ds(row0, CHUNK)],
                                     wsem[wslot])
        nt = t + NG
        if nt < NT:
            start_gather(nt)

    for d in wd:
        if d is not None:
            d.wait()


def kernel(x, table):
    cst = jnp.asarray(_POS_CST)

    mesh = plsc.VectorSubcoreMesh(
        core_axis_name="c", subcore_axis_name="s", num_cores=NC, num_subcores=NS
    )
    k = pl.kernel(
        _sc_body,
        out_type=jax.ShapeDtypeStruct((B * L, D), jnp.float32),
        mesh=mesh,
        scratch_types=[
            pltpu.VMEM((B * P_PER_W,), jnp.int32),
            pltpu.VMEM((P_PER_W, D), jnp.float32),
            pltpu.VMEM((1, D), jnp.float32),
            pltpu.VMEM((NCHAIN, D), jnp.float32),
            pltpu.VMEM((CHUNK, D), jnp.float32),
            pltpu.VMEM((CHUNK, D), jnp.float32),
            pltpu.VMEM((CHUNK, D), jnp.float32),
            pltpu.VMEM((CHUNK, D), jnp.float32),
            pltpu.VMEM((CHUNK, D), jnp.float32),
            pltpu.VMEM((CHUNK, D), jnp.float32),
            pltpu.SemaphoreType.DMA,
            pltpu.SemaphoreType.DMA,
            pltpu.SemaphoreType.DMA,
            pltpu.SemaphoreType.DMA,
            pltpu.SemaphoreType.DMA,
            pltpu.SemaphoreType.DMA,
            pltpu.SemaphoreType.DMA,
            pltpu.SemaphoreType.DMA,
            pltpu.SemaphoreType.DMA,
            pltpu.SemaphoreType.DMA,
        ],
    )
    out = k(x, table, cst)
    return out.reshape(B, L, D)
